# Initial kernel scaffold; baseline (speedup 1.0000x reference)
import jax
import jax.numpy as jnp
from jax.experimental import pallas as pl


def kernel(query_features, memory, k):
    raise NotImplementedError("write your pallas kernel here")


# speedup vs baseline: 2.4323x; 2.4323x over previous
import functools

import jax
import jax.numpy as jnp
from jax import lax
from jax.experimental import pallas as pl
from jax.experimental.pallas import tpu as pltpu
from jax.experimental.pallas import tpu_sc as plsc

Q = 1024
D = 128
M = 100000
K = 5
TILE_M = 2048
NEG_INF = float("-inf")
INT_MAX = 2**31 - 1


def _topk_body(q_ref, mem_ref, outi_ref, rv, ri):
    i = pl.program_id(0)
    nt = pl.num_programs(0)

    @pl.when(i == 0)
    def _():
        rv[...] = jnp.full((Q, 128), NEG_INF, jnp.float32)
        ri[...] = jnp.full((Q, 128), -1, jnp.int32)

    q = q_ref[...]
    qn = q / jnp.maximum(jnp.sqrt(jnp.sum(q * q, axis=1, keepdims=True)), 1e-8)
    m = mem_ref[...]
    mn = m / jnp.maximum(jnp.sqrt(jnp.sum(m * m, axis=1, keepdims=True)), 1e-8)
    sims = lax.dot_general(
        qn, mn, (((1,), (1,)), ((), ())), preferred_element_type=jnp.float32
    )
    gidx = i * TILE_M + lax.broadcasted_iota(jnp.int32, (Q, TILE_M), 1)
    sims = jnp.where(gidx < M, sims, NEG_INF)

    cv = jnp.concatenate([rv[...], sims], axis=1)
    ci = jnp.concatenate([ri[...], gidx], axis=1)
    vals, idxs = [], []
    for _ in range(K):
        mx = jnp.max(cv, axis=1, keepdims=True)
        sel = jnp.min(jnp.where(cv == mx, ci, INT_MAX), axis=1, keepdims=True)
        vals.append(mx)
        idxs.append(sel)
        cv = jnp.where(ci == sel, NEG_INF, cv)

    lane = lax.broadcasted_iota(jnp.int32, (Q, 128), 1)
    nrv = jnp.full((Q, 128), NEG_INF, jnp.float32)
    nri = jnp.full((Q, 128), -1, jnp.int32)
    for j in range(K):
        nrv = jnp.where(lane == j, vals[j], nrv)
        nri = jnp.where(lane == j, idxs[j], nri)
    rv[...] = nrv
    ri[...] = nri

    @pl.when(i == nt - 1)
    def _():
        outi_ref[...] = nri


_topk_call = pl.pallas_call(
    _topk_body,
    grid=((M + TILE_M - 1) // TILE_M,),
    in_specs=[
        pl.BlockSpec((Q, D), lambda i: (0, 0)),
        pl.BlockSpec((TILE_M, D), lambda i: (i, 0)),
    ],
    out_specs=pl.BlockSpec((Q, 128), lambda i: (0, 0)),
    out_shape=jax.ShapeDtypeStruct((Q, 128), jnp.int32),
    scratch_shapes=[
        pltpu.VMEM((Q, 128), jnp.float32),
        pltpu.VMEM((Q, 128), jnp.int32),
    ],
)


@functools.cache
def _gather_mean():
    info = plsc.get_sparse_core_info()
    nw = info.num_cores * info.num_subcores
    bpw = (Q * K) // nw
    n_ch = -(-bpw // 128)
    bpc = bpw // n_ch
    qpw = Q // nw

    mesh = plsc.VectorSubcoreMesh(core_axis_name="c", subcore_axis_name="s")

    @functools.partial(
        pl.kernel,
        mesh=mesh,
        out_type=jax.ShapeDtypeStruct((Q, D), jnp.float32),
        scratch_types=[
            pltpu.VMEM((bpw,), jnp.int32),
            pltpu.VMEM((bpw, D), jnp.float32),
            pltpu.VMEM((qpw, D), jnp.float32),
            pltpu.SemaphoreType.DMA,
        ],
    )
    def gather_mean(mem_hbm, idx_hbm, out_hbm, idx_v, rows_v, out_v, sem):
        wid = lax.axis_index("s") * info.num_cores + lax.axis_index("c")
        base = wid * bpw
        pltpu.sync_copy(idx_hbm.at[pl.ds(base, bpw)], idx_v)
        copies = [
            pltpu.async_copy(
                mem_hbm.at[idx_v.at[pl.ds(c * bpc, bpc)]],
                rows_v.at[pl.ds(c * bpc, bpc)],
                sem,
            )
            for c in range(n_ch)
        ]
        for cp in copies:
            cp.wait()

        def qbody(qi, carry):
            for dblk in range(D // 16):
                sl = pl.ds(dblk * 16, 16)
                acc = rows_v[qi * K, sl]
                for j in range(1, K):
                    acc = acc + rows_v[qi * K + j, sl]
                out_v[qi, sl] = acc
            return carry

        lax.fori_loop(0, qpw, qbody, 0)
        pltpu.sync_copy(out_v, out_hbm.at[pl.ds(wid * qpw, qpw)])

    return gather_mean


def kernel(query_features, memory, k):
    top_idx = _topk_call(query_features, memory)
    idx_flat = top_idx[:, :K].reshape(-1)
    sums = _gather_mean()(memory, idx_flat)
    return sums / k


# speedup vs baseline: 3.0472x; 1.2528x over previous
import functools

import jax
import jax.numpy as jnp
from jax import lax
from jax.experimental import pallas as pl
from jax.experimental.pallas import tpu as pltpu
from jax.experimental.pallas import tpu_sc as plsc

Q = 1024
D = 128
M = 100000
K = 5
TILE_M = 2048
NEG_INF = float("-inf")
INT_MAX = 2**31 - 1

_GRID = (M + TILE_M - 1) // TILE_M
assert (_GRID - 1) * TILE_M + TILE_M // 2 <= M


def _topk_body(q_ref, mem_ref, outi_ref, rv, ri):
    i = pl.program_id(0)
    nt = pl.num_programs(0)

    @pl.when(i == 0)
    def _():
        rv[...] = jnp.full((Q, 128), NEG_INF, jnp.float32)
        ri[...] = jnp.full((Q, 128), -1.0, jnp.float32)

    q = q_ref[...]
    qn = q / jnp.maximum(jnp.sqrt(jnp.sum(q * q, axis=1, keepdims=True)), 1e-8)
    m = mem_ref[...]
    mn = m / jnp.maximum(jnp.sqrt(jnp.sum(m * m, axis=1, keepdims=True)), 1e-8)
    sims = lax.dot_general(
        qn, mn, (((1,), (1,)), ((), ())), preferred_element_type=jnp.float32
    )

    half = TILE_M // 2
    a = sims[:, :half]
    b = sims[:, half:]
    ia = jnp.float32(i * TILE_M) + lax.broadcasted_iota(
        jnp.int32, (Q, half), 1
    ).astype(jnp.float32)
    ib = ia + jnp.float32(half)
    b = jnp.where(ib < jnp.float32(M), b, NEG_INF)
    ge = a >= b
    h = jnp.where(ge, a, b)
    lo = jnp.where(ge, b, a)
    hi = jnp.where(ge, ia, ib)
    li = jnp.where(ge, ib, ia)

    cv = jnp.concatenate([rv[...], h], axis=1)
    ci = jnp.concatenate([ri[...], hi], axis=1)
    lw = jnp.concatenate([jnp.full((Q, 128), NEG_INF, jnp.float32), lo], axis=1)
    lx = jnp.concatenate([jnp.full((Q, 128), -1.0, jnp.float32), li], axis=1)
    vals, idxs = [], []
    for j in range(K):
        mx = jnp.max(cv, axis=1, keepdims=True)
        sel = jnp.min(jnp.where(cv == mx, ci, jnp.inf), axis=1, keepdims=True)
        vals.append(mx)
        idxs.append(sel)
        if j < K - 1:
            hit = ci == sel
            cv = jnp.where(hit, lw, cv)
            ci = jnp.where(hit, lx, ci)
            lw = jnp.where(hit, NEG_INF, lw)

    lane = lax.broadcasted_iota(jnp.int32, (Q, 128), 1)
    nrv = jnp.full((Q, 128), NEG_INF, jnp.float32)
    nri = jnp.full((Q, 128), -1.0, jnp.float32)
    for j in range(K):
        nrv = jnp.where(lane == j, vals[j], nrv)
        nri = jnp.where(lane == j, idxs[j], nri)
    rv[...] = nrv
    ri[...] = nri

    @pl.when(i == nt - 1)
    def _():
        outi_ref[...] = nri.astype(jnp.int32)


_topk_call = pl.pallas_call(
    _topk_body,
    grid=(_GRID,),
    in_specs=[
        pl.BlockSpec((Q, D), lambda i: (0, 0)),
        pl.BlockSpec((TILE_M, D), lambda i: (i, 0)),
    ],
    out_specs=pl.BlockSpec((Q, 128), lambda i: (0, 0)),
    out_shape=jax.ShapeDtypeStruct((Q, 128), jnp.int32),
    scratch_shapes=[
        pltpu.VMEM((Q, 128), jnp.float32),
        pltpu.VMEM((Q, 128), jnp.float32),
    ],
)


@functools.cache
def _gather_mean():
    info = plsc.get_sparse_core_info()
    nw = info.num_cores * info.num_subcores
    bpw = (Q * K) // nw
    n_ch = -(-bpw // 128)
    bpc = bpw // n_ch
    qpw = Q // nw

    mesh = plsc.VectorSubcoreMesh(core_axis_name="c", subcore_axis_name="s")

    @functools.partial(
        pl.kernel,
        mesh=mesh,
        out_type=jax.ShapeDtypeStruct((Q, D), jnp.float32),
        scratch_types=[
            pltpu.VMEM((bpw,), jnp.int32),
            pltpu.VMEM((bpw, D), jnp.float32),
            pltpu.VMEM((qpw, D), jnp.float32),
            pltpu.SemaphoreType.DMA,
        ],
    )
    def gather_mean(mem_hbm, idx_hbm, out_hbm, idx_v, rows_v, out_v, sem):
        wid = lax.axis_index("s") * info.num_cores + lax.axis_index("c")
        base = wid * bpw
        pltpu.sync_copy(idx_hbm.at[pl.ds(base, bpw)], idx_v)
        copies = [
            pltpu.async_copy(
                mem_hbm.at[idx_v.at[pl.ds(c * bpc, bpc)]],
                rows_v.at[pl.ds(c * bpc, bpc)],
                sem,
            )
            for c in range(n_ch)
        ]
        for cp in copies:
            cp.wait()

        def qbody(qi, carry):
            for dblk in range(D // 16):
                sl = pl.ds(dblk * 16, 16)
                acc = rows_v[qi * K, sl]
                for j in range(1, K):
                    acc = acc + rows_v[qi * K + j, sl]
                out_v[qi, sl] = acc
            return carry

        lax.fori_loop(0, qpw, qbody, 0)
        pltpu.sync_copy(out_v, out_hbm.at[pl.ds(wid * qpw, qpw)])

    return gather_mean


def kernel(query_features, memory, k):
    top_idx = _topk_call(query_features, memory)
    idx_flat = top_idx[:, :K].reshape(-1)
    sums = _gather_mean()(memory, idx_flat)
    return sums / k
